# baseline (speedup 1.0000x reference)
import functools

import jax
import jax.numpy as jnp
from jax import lax
from jax.experimental import pallas as pl
from jax.experimental.pallas import tpu as pltpu
from jax.experimental.pallas import tpu_sc as plsc

_NUM_CORES = 2
_NUM_SUBCORES = 16
_LANES = 16
_NUM_WORKERS = _NUM_CORES * _NUM_SUBCORES

_LATENT = 64
_PAIR = 2 * _LATENT
_CHUNK = 128


def _perm(x, idx):
    return lax.gather(
        x, idx[:, None],
        dimension_numbers=lax.GatherDimensionNumbers(
            offset_dims=(), collapsed_slice_dims=(0,), start_index_map=(0,)),
        slice_sizes=(1,),
        mode=lax.GatherScatterMode.PROMISE_IN_BOUNDS)


def _make_kernel(batch: int):
    b_per_w = batch // _NUM_WORKERS
    n_chunks = b_per_w // _CHUNK
    groups_per_chunk = _CHUNK // _LANES

    mesh = plsc.VectorSubcoreMesh(
        core_axis_name="c",
        subcore_axis_name="s",
        num_cores=_NUM_CORES,
        num_subcores=_NUM_SUBCORES,
    )

    @functools.partial(
        pl.kernel,
        mesh=mesh,
        out_type=jax.ShapeDtypeStruct((batch,), jnp.float32),
        scratch_types=[
            pltpu.VMEM((b_per_w,), jnp.int32),
            pltpu.VMEM((b_per_w,), jnp.int32),
            pltpu.VMEM((b_per_w,), jnp.int32),
            pltpu.VMEM((b_per_w,), jnp.int32),
            pltpu.VMEM((2, _CHUNK, 128), jnp.int32),
            pltpu.VMEM((2, _CHUNK, 128), jnp.int32),
            pltpu.VMEM((b_per_w,), jnp.float32),
            pltpu.SemaphoreType.DMA,
        ],
        compiler_params=pltpu.CompilerParams(needs_layout_passes=False),
    )
    def k(urow_hbm, prow_hbm, uoff_hbm, poff_hbm, ut_hbm, pt_hbm, out_hbm,
          urow_v, prow_v, uoff_v, poff_v, urows, prows, outv, sem):
        wid = lax.axis_index("s") * _NUM_CORES + lax.axis_index("c")
        base = wid * b_per_w

        pltpu.sync_copy(urow_hbm.at[pl.ds(base, b_per_w)], urow_v)
        pltpu.sync_copy(prow_hbm.at[pl.ds(base, b_per_w)], prow_v)
        pltpu.sync_copy(uoff_hbm.at[pl.ds(base, b_per_w)], uoff_v)
        pltpu.sync_copy(poff_hbm.at[pl.ds(base, b_per_w)], poff_v)

        lane = lax.iota(jnp.int32, _LANES)

        def fire(c):
            slot = c % 2
            pltpu.async_copy(
                ut_hbm.at[urow_v.at[pl.ds(c * _CHUNK, _CHUNK)]],
                urows.at[slot], sem)
            pltpu.async_copy(
                pt_hbm.at[prow_v.at[pl.ds(c * _CHUNK, _CHUNK)]],
                prows.at[slot], sem)

        def drain(c):
            slot = c % 2
            pltpu.make_async_copy(
                ut_hbm.at[pl.ds(0, _CHUNK)], urows.at[slot], sem).wait()
            pltpu.make_async_copy(
                pt_hbm.at[pl.ds(0, _CHUNK)], prows.at[slot], sem).wait()

        def compute(c):
            slot = c % 2

            def group_body(g, _):
                b0 = c * _CHUNK + g * _LANES
                uo_vec = uoff_v[pl.ds(b0, _LANES)]
                po_vec = poff_v[pl.ds(b0, _LANES)]
                res = jnp.zeros((_LANES,), jnp.float32)
                for j in range(_LANES):
                    r = g * _LANES + j
                    uo = pl.multiple_of(uo_vec[j], 32)
                    po = pl.multiple_of(po_vec[j], 32)
                    acc = jnp.zeros((_LANES,), jnp.float32)
                    for v in range(2):
                        u2 = plsc.bitcast(
                            urows[slot, r, pl.ds(uo + v * _LANES, _LANES)],
                            jnp.bfloat16)
                        p2 = plsc.bitcast(
                            prows[slot, r, pl.ds(po + v * _LANES, _LANES)],
                            jnp.bfloat16)
                        ua, ub = plsc.unpack(
                            u2, format=plsc.PackFormat.INTERLEAVED)
                        pa, pb = plsc.unpack(
                            p2, format=plsc.PackFormat.INTERLEAVED)
                        acc = acc + ua * pa + ub * pb
                    for step in (8, 4, 2, 1):
                        acc = acc + _perm(acc, lane ^ step)
                    res = jnp.where(lane == j, acc, res)
                res = 1.0 / (1.0 + jnp.exp(-res))
                outv[pl.ds(b0, _LANES)] = res
                return 0

            lax.fori_loop(0, groups_per_chunk, group_body, 0)

        fire(0)

        def pipe_body(c, _):
            fire(c)
            drain(c - 1)
            compute(c - 1)
            return 0

        lax.fori_loop(1, n_chunks, pipe_body, 0)
        drain(n_chunks - 1)
        compute(n_chunks - 1)

        pltpu.sync_copy(outv, out_hbm.at[pl.ds(base, b_per_w)])

    return k


@jax.jit
def kernel(inputs, user_table, product_table):
    batch = inputs.shape[0]
    uidx = inputs[:, 0].astype(jnp.int32)
    pidx = inputs[:, 1].astype(jnp.int32)
    urow = uidx >> 2
    prow = pidx >> 2
    uoff = (uidx & 3) * 32
    poff = (pidx & 3) * 32
    ut2 = lax.bitcast_convert_type(
        user_table.astype(jnp.bfloat16).reshape(-1, _LATENT // 2, 2),
        jnp.int32).reshape(user_table.shape[0] // 4, 128)
    pt2 = lax.bitcast_convert_type(
        product_table.astype(jnp.bfloat16).reshape(-1, _LATENT // 2, 2),
        jnp.int32).reshape(product_table.shape[0] // 4, 128)
    k = _make_kernel(batch)
    return k(urow, prow, uoff, poff, ut2, pt2)


# speedup vs baseline: 7.1962x; 7.1962x over previous
import functools

import jax
import jax.numpy as jnp
from jax import lax
from jax.experimental import pallas as pl
from jax.experimental.pallas import tpu as pltpu
from jax.experimental.pallas import tpu_sc as plsc

_NUM_CORES = 2
_NUM_SUBCORES = 16
_LANES = 16
_NUM_WORKERS = _NUM_CORES * _NUM_SUBCORES

_LATENT = 64
_BLK = 128


def _perm(x, idx):
    return lax.gather(
        x, idx[:, None],
        dimension_numbers=lax.GatherDimensionNumbers(
            offset_dims=(), collapsed_slice_dims=(0,), start_index_map=(0,)),
        slice_sizes=(1,),
        mode=lax.GatherScatterMode.PROMISE_IN_BOUNDS)


def _make_kernel(batch: int, rows: int):
    b_per_w = batch // _NUM_WORKERS
    n_groups = b_per_w // _LANES
    last_full = ((rows // _BLK) - 0) * _BLK - _BLK
    tail_start = (rows // _BLK) * _BLK

    mesh = plsc.VectorSubcoreMesh(
        core_axis_name="c",
        subcore_axis_name="s",
        num_cores=_NUM_CORES,
        num_subcores=_NUM_SUBCORES,
    )

    @functools.partial(
        pl.kernel,
        mesh=mesh,
        out_type=jax.ShapeDtypeStruct((batch,), jnp.float32),
        scratch_types=[
            pltpu.VMEM((b_per_w,), jnp.int32),
            pltpu.VMEM((b_per_w,), jnp.int32),
            pltpu.VMEM((4, _LATENT, _BLK), jnp.float32),
            pltpu.VMEM((4, _LATENT, _BLK), jnp.float32),
            pltpu.VMEM((_LATENT, _BLK), jnp.float32),
            pltpu.VMEM((_LATENT, _BLK), jnp.float32),
            pltpu.VMEM((b_per_w,), jnp.float32),
            pltpu.SemaphoreType.DMA,
            pltpu.SemaphoreType.DMA,
        ],
        compiler_params=pltpu.CompilerParams(needs_layout_passes=False),
    )
    def k(uidx_hbm, pidx_hbm, ut_hbm, pt_hbm, utail_hbm, ptail_hbm, out_hbm,
          uidx_s, pidx_s, ublk, pblk, utail, ptail, outv, usem, psem):
        wid = lax.axis_index("s") * _NUM_CORES + lax.axis_index("c")
        base = wid * b_per_w

        pltpu.sync_copy(uidx_hbm.at[pl.ds(base, b_per_w)], uidx_s)
        pltpu.sync_copy(pidx_hbm.at[pl.ds(base, b_per_w)], pidx_s)
        pltpu.sync_copy(utail_hbm, utail)
        pltpu.sync_copy(ptail_hbm, ptail)

        lane = lax.iota(jnp.int32, _LANES)

        def group_body(g, _):
            uvec = uidx_s[pl.ds(g * _LANES, _LANES)]
            pvec = pidx_s[pl.ds(g * _LANES, _LANES)]
            ucb_all = jnp.minimum((uvec >> 7) << 7,
                                  jnp.full((_LANES,), last_full, jnp.int32))
            pcb_all = jnp.minimum((pvec >> 7) << 7,
                                  jnp.full((_LANES,), last_full, jnp.int32))
            ucol_all = jnp.minimum(uvec - ucb_all,
                                   jnp.full((_LANES,), _BLK - 1, jnp.int32))
            pcol_all = jnp.minimum(pvec - pcb_all,
                                   jnp.full((_LANES,), _BLK - 1, jnp.int32))
            utcol_all = jnp.clip(uvec - tail_start, 0, _LATENT - 1)
            ptcol_all = jnp.clip(pvec - tail_start, 0, _LATENT - 1)

            def fire(j):
                slot = j % 4
                cu = pl.multiple_of(
                    jnp.minimum((uvec[j] >> 7) << 7, last_full), _BLK)
                cp = pl.multiple_of(
                    jnp.minimum((pvec[j] >> 7) << 7, last_full), _BLK)
                pltpu.async_copy(
                    ut_hbm.at[:, pl.ds(cu, _BLK)], ublk.at[slot], usem)
                pltpu.async_copy(
                    pt_hbm.at[:, pl.ds(cp, _BLK)], pblk.at[slot], psem)

            def drain(j):
                slot = j % 4
                pltpu.make_async_copy(
                    ut_hbm.at[:, pl.ds(0, _BLK)], ublk.at[slot], usem).wait()
                pltpu.make_async_copy(
                    pt_hbm.at[:, pl.ds(0, _BLK)], pblk.at[slot], psem).wait()

            def compute(j, res):
                slot = j % 4
                jf = jnp.full((_LANES,), j, jnp.int32)
                ucol = _perm(ucol_all, jf)
                pcol = _perm(pcol_all, jf)
                utc = _perm(utcol_all, jf)
                ptc = _perm(ptcol_all, jf)
                tacc = jnp.zeros((_LANES,), jnp.float32)
                for v in range(_LATENT // _LANES):
                    dvec = v * _LANES + lane
                    u = plsc.load_gather(ublk.at[slot], [dvec, ucol])
                    p = plsc.load_gather(pblk.at[slot], [dvec, pcol])
                    tu = plsc.load_gather(utail, [dvec, utc])
                    tp = plsc.load_gather(ptail, [dvec, ptc])
                    tacc = tacc + jnp.where(
                        _perm(uvec, jf) >= tail_start, tu, u) * jnp.where(
                        _perm(pvec, jf) >= tail_start, tp, p)
                for step in (8, 4, 2, 1):
                    tacc = tacc + _perm(tacc, lane ^ step)
                return jnp.where(lane == j, tacc, res)

            res = jnp.zeros((_LANES,), jnp.float32)
            fire(0)
            fire(1)
            for j in range(_LANES):
                if j + 2 < _LANES:
                    fire(j + 2)
                drain(j)
                res = compute(j, res)
            res = 1.0 / (1.0 + jnp.exp(-res))
            outv[pl.ds(g * _LANES, _LANES)] = res
            return 0

        lax.fori_loop(0, n_groups, group_body, 0)

        pltpu.sync_copy(outv, out_hbm.at[pl.ds(base, b_per_w)])

    return k


@jax.jit
def kernel(inputs, user_table, product_table):
    batch = inputs.shape[0]
    rows = user_table.shape[0]
    uidx = inputs[:, 0].astype(jnp.int32)
    pidx = inputs[:, 1].astype(jnp.int32)
    ut_t = user_table.T
    pt_t = product_table.T
    tail_start = (rows // _BLK) * _BLK
    pad = _BLK - (rows - tail_start)
    ut_tail = jnp.pad(ut_t[:, tail_start:], ((0, 0), (0, pad)))
    pt_tail = jnp.pad(pt_t[:, tail_start:], ((0, 0), (0, pad)))
    k = _make_kernel(batch, rows)
    return k(uidx, pidx, ut_t, pt_t, ut_tail, pt_tail)


# speedup vs baseline: 7.4262x; 1.0320x over previous
import functools

import jax
import jax.numpy as jnp
from jax import lax
from jax.experimental import pallas as pl
from jax.experimental.pallas import tpu as pltpu
from jax.experimental.pallas import tpu_sc as plsc

_NUM_CORES = 2
_NUM_SUBCORES = 16
_LANES = 16
_NUM_WORKERS = _NUM_CORES * _NUM_SUBCORES

_LATENT = 64
_BLK = 128


def _perm(x, idx):
    return lax.gather(
        x, idx[:, None],
        dimension_numbers=lax.GatherDimensionNumbers(
            offset_dims=(), collapsed_slice_dims=(0,), start_index_map=(0,)),
        slice_sizes=(1,),
        mode=lax.GatherScatterMode.PROMISE_IN_BOUNDS)


def _make_kernel(batch: int, rows: int):
    b_per_w = batch // _NUM_WORKERS
    n_groups = b_per_w // _LANES
    last_full = ((rows // _BLK) - 0) * _BLK - _BLK
    tail_start = (rows // _BLK) * _BLK

    mesh = plsc.VectorSubcoreMesh(
        core_axis_name="c",
        subcore_axis_name="s",
        num_cores=_NUM_CORES,
        num_subcores=_NUM_SUBCORES,
    )

    @functools.partial(
        pl.kernel,
        mesh=mesh,
        out_type=jax.ShapeDtypeStruct((batch,), jnp.float32),
        scratch_types=[
            pltpu.VMEM((b_per_w,), jnp.int32),
            pltpu.VMEM((b_per_w,), jnp.int32),
            pltpu.VMEM((4, _LATENT, _BLK), jnp.float32),
            pltpu.VMEM((4, _LATENT, _BLK), jnp.float32),
            pltpu.VMEM((_LATENT, _BLK), jnp.float32),
            pltpu.VMEM((_LATENT, _BLK), jnp.float32),
            pltpu.VMEM((b_per_w,), jnp.float32),
            pltpu.SemaphoreType.DMA,
            pltpu.SemaphoreType.DMA,
        ],
        compiler_params=pltpu.CompilerParams(needs_layout_passes=False),
    )
    def k(uidx_hbm, pidx_hbm, ut_hbm, pt_hbm, utail_hbm, ptail_hbm, out_hbm,
          uidx_s, pidx_s, ublk, pblk, utail, ptail, outv, usem, psem):
        wid = lax.axis_index("s") * _NUM_CORES + lax.axis_index("c")
        base = wid * b_per_w

        pltpu.sync_copy(uidx_hbm.at[pl.ds(base, b_per_w)], uidx_s)
        pltpu.sync_copy(pidx_hbm.at[pl.ds(base, b_per_w)], pidx_s)
        pltpu.sync_copy(utail_hbm, utail)
        pltpu.sync_copy(ptail_hbm, ptail)

        lane = lax.iota(jnp.int32, _LANES)

        def fire_vec(uvec, pvec, j, slot):
            cu = pl.multiple_of(
                jnp.minimum((uvec[j] >> 7) << 7, last_full), _BLK)
            cp = pl.multiple_of(
                jnp.minimum((pvec[j] >> 7) << 7, last_full), _BLK)
            pltpu.async_copy(
                ut_hbm.at[:, pl.ds(cu, _BLK)], ublk.at[slot], usem)
            pltpu.async_copy(
                pt_hbm.at[:, pl.ds(cp, _BLK)], pblk.at[slot], psem)

        uvec0 = uidx_s[pl.ds(0, _LANES)]
        pvec0 = pidx_s[pl.ds(0, _LANES)]
        fire_vec(uvec0, pvec0, 0, 0)
        fire_vec(uvec0, pvec0, 1, 1)

        def group_body(g, _):
            uvec = uidx_s[pl.ds(g * _LANES, _LANES)]
            pvec = pidx_s[pl.ds(g * _LANES, _LANES)]
            ucb_all = jnp.minimum((uvec >> 7) << 7,
                                  jnp.full((_LANES,), last_full, jnp.int32))
            pcb_all = jnp.minimum((pvec >> 7) << 7,
                                  jnp.full((_LANES,), last_full, jnp.int32))
            ucol_all = jnp.minimum(uvec - ucb_all,
                                   jnp.full((_LANES,), _BLK - 1, jnp.int32))
            pcol_all = jnp.minimum(pvec - pcb_all,
                                   jnp.full((_LANES,), _BLK - 1, jnp.int32))
            utcol_all = jnp.clip(uvec - tail_start, 0, _LATENT - 1)
            ptcol_all = jnp.clip(pvec - tail_start, 0, _LATENT - 1)

            def drain(j):
                slot = j % 4
                pltpu.make_async_copy(
                    ut_hbm.at[:, pl.ds(0, _BLK)], ublk.at[slot], usem).wait()
                pltpu.make_async_copy(
                    pt_hbm.at[:, pl.ds(0, _BLK)], pblk.at[slot], psem).wait()

            def compute(j, res):
                slot = j % 4
                jf = jnp.full((_LANES,), j, jnp.int32)
                ucol = _perm(ucol_all, jf)
                pcol = _perm(pcol_all, jf)
                utc = _perm(utcol_all, jf)
                ptc = _perm(ptcol_all, jf)
                tacc = jnp.zeros((_LANES,), jnp.float32)
                for v in range(_LATENT // _LANES):
                    dvec = v * _LANES + lane
                    u = plsc.load_gather(ublk.at[slot], [dvec, ucol])
                    p = plsc.load_gather(pblk.at[slot], [dvec, pcol])
                    tu = plsc.load_gather(utail, [dvec, utc])
                    tp = plsc.load_gather(ptail, [dvec, ptc])
                    tacc = tacc + jnp.where(
                        _perm(uvec, jf) >= tail_start, tu, u) * jnp.where(
                        _perm(pvec, jf) >= tail_start, tp, p)
                for step in (8, 4, 2, 1):
                    tacc = tacc + _perm(tacc, lane ^ step)
                return jnp.where(lane == j, tacc, res)

            res = jnp.zeros((_LANES,), jnp.float32)
            for j in range(_LANES):
                if j + 2 < _LANES:
                    fire_vec(uvec, pvec, j + 2, (j + 2) % 4)
                else:
                    jn = j + 2 - _LANES

                    @pl.when(g + 1 < n_groups)
                    def _():
                        uvec_n = uidx_s[pl.ds((g + 1) * _LANES, _LANES)]
                        pvec_n = pidx_s[pl.ds((g + 1) * _LANES, _LANES)]
                        fire_vec(uvec_n, pvec_n, jn, jn)

                drain(j)
                res = compute(j, res)
            res = 1.0 / (1.0 + jnp.exp(-res))
            outv[pl.ds(g * _LANES, _LANES)] = res
            return 0

        lax.fori_loop(0, n_groups, group_body, 0)

        pltpu.sync_copy(outv, out_hbm.at[pl.ds(base, b_per_w)])

    return k


@jax.jit
def kernel(inputs, user_table, product_table):
    batch = inputs.shape[0]
    rows = user_table.shape[0]
    uidx = inputs[:, 0].astype(jnp.int32)
    pidx = inputs[:, 1].astype(jnp.int32)
    ut_t = user_table.T
    pt_t = product_table.T
    tail_start = (rows // _BLK) * _BLK
    pad = _BLK - (rows - tail_start)
    ut_tail = jnp.pad(ut_t[:, tail_start:], ((0, 0), (0, pad)))
    pt_tail = jnp.pad(pt_t[:, tail_start:], ((0, 0), (0, pad)))
    k = _make_kernel(batch, rows)
    return k(uidx, pidx, ut_t, pt_t, ut_tail, pt_tail)


# speedup vs baseline: 7.4329x; 1.0009x over previous
import functools

import jax
import jax.numpy as jnp
from jax import lax
from jax.experimental import pallas as pl
from jax.experimental.pallas import tpu as pltpu
from jax.experimental.pallas import tpu_sc as plsc

_NUM_CORES = 2
_NUM_SUBCORES = 16
_LANES = 16
_NUM_WORKERS = _NUM_CORES * _NUM_SUBCORES

_LATENT = 64
_BLK = 128


def _perm(x, idx):
    return lax.gather(
        x, idx[:, None],
        dimension_numbers=lax.GatherDimensionNumbers(
            offset_dims=(), collapsed_slice_dims=(0,), start_index_map=(0,)),
        slice_sizes=(1,),
        mode=lax.GatherScatterMode.PROMISE_IN_BOUNDS)


def _make_kernel(batch: int, rows: int):
    b_per_w = batch // _NUM_WORKERS
    n_groups = b_per_w // _LANES
    last_full = (rows // _BLK) * _BLK - _BLK
    tail_start = (rows // _BLK) * _BLK

    mesh = plsc.VectorSubcoreMesh(
        core_axis_name="c",
        subcore_axis_name="s",
        num_cores=_NUM_CORES,
        num_subcores=_NUM_SUBCORES,
    )

    @functools.partial(
        pl.kernel,
        mesh=mesh,
        out_type=jax.ShapeDtypeStruct((batch,), jnp.float32),
        scratch_types=[
            pltpu.VMEM((b_per_w,), jnp.int32),
            pltpu.VMEM((b_per_w,), jnp.int32),
            pltpu.VMEM((4, _LATENT, _BLK), jnp.float32),
            pltpu.VMEM((4, _LATENT, _BLK), jnp.float32),
            pltpu.VMEM((_LATENT, _BLK), jnp.float32),
            pltpu.VMEM((_LATENT, _BLK), jnp.float32),
            pltpu.VMEM((b_per_w,), jnp.float32),
            pltpu.SemaphoreType.DMA,
            pltpu.SemaphoreType.DMA,
        ],
        compiler_params=pltpu.CompilerParams(needs_layout_passes=False),
    )
    def k(uidx_hbm, pidx_hbm, ut_hbm, pt_hbm, utail_hbm, ptail_hbm, out_hbm,
          uidx_s, pidx_s, ublk, pblk, utail, ptail, outv, usem, psem):
        wid = lax.axis_index("s") * _NUM_CORES + lax.axis_index("c")
        base = wid * b_per_w

        pltpu.sync_copy(uidx_hbm.at[pl.ds(base, b_per_w)], uidx_s)
        pltpu.sync_copy(pidx_hbm.at[pl.ds(base, b_per_w)], pidx_s)
        pltpu.sync_copy(utail_hbm, utail)
        pltpu.sync_copy(ptail_hbm, ptail)

        lane = lax.iota(jnp.int32, _LANES)

        def fire_vec(uvec, pvec, j, slot):
            cu = pl.multiple_of(
                jnp.minimum((uvec[j] >> 7) << 7, last_full), _BLK)
            cp = pl.multiple_of(
                jnp.minimum((pvec[j] >> 7) << 7, last_full), _BLK)
            pltpu.async_copy(
                ut_hbm.at[:, pl.ds(cu, _BLK)], ublk.at[slot], usem)
            pltpu.async_copy(
                pt_hbm.at[:, pl.ds(cp, _BLK)], pblk.at[slot], psem)

        uvec0 = uidx_s[pl.ds(0, _LANES)]
        pvec0 = pidx_s[pl.ds(0, _LANES)]
        fire_vec(uvec0, pvec0, 0, 0)
        fire_vec(uvec0, pvec0, 1, 1)

        def group_body(g, _):
            uvec = uidx_s[pl.ds(g * _LANES, _LANES)]
            pvec = pidx_s[pl.ds(g * _LANES, _LANES)]
            ucb_all = jnp.minimum((uvec >> 7) << 7,
                                  jnp.full((_LANES,), last_full, jnp.int32))
            pcb_all = jnp.minimum((pvec >> 7) << 7,
                                  jnp.full((_LANES,), last_full, jnp.int32))
            ucol_all = jnp.minimum(uvec - ucb_all,
                                   jnp.full((_LANES,), _BLK - 1, jnp.int32))
            pcol_all = jnp.minimum(pvec - pcb_all,
                                   jnp.full((_LANES,), _BLK - 1, jnp.int32))
            utcol_all = jnp.clip(uvec - tail_start, 0, _LATENT - 1)
            ptcol_all = jnp.clip(pvec - tail_start, 0, _LATENT - 1)

            def drain(j):
                slot = j % 4
                pltpu.make_async_copy(
                    ut_hbm.at[:, pl.ds(0, _BLK)], ublk.at[slot], usem).wait()
                pltpu.make_async_copy(
                    pt_hbm.at[:, pl.ds(0, _BLK)], pblk.at[slot], psem).wait()

            def compute(j, res):
                slot = j % 4
                jf = jnp.full((_LANES,), j, jnp.int32)
                ucol = _perm(ucol_all, jf)
                pcol = _perm(pcol_all, jf)
                utc = _perm(utcol_all, jf)
                ptc = _perm(ptcol_all, jf)
                tacc = jnp.zeros((_LANES,), jnp.float32)
                for v in range(_LATENT // _LANES):
                    dvec = v * _LANES + lane
                    u = plsc.load_gather(ublk.at[slot], [dvec, ucol])
                    p = plsc.load_gather(pblk.at[slot], [dvec, pcol])
                    tu = plsc.load_gather(utail, [dvec, utc])
                    tp = plsc.load_gather(ptail, [dvec, ptc])
                    tacc = tacc + jnp.where(
                        _perm(uvec, jf) >= tail_start, tu, u) * jnp.where(
                        _perm(pvec, jf) >= tail_start, tp, p)
                for step in (8, 4, 2, 1):
                    tacc = tacc + _perm(tacc, lane ^ step)
                return jnp.where(lane == j, tacc, res)

            res = jnp.zeros((_LANES,), jnp.float32)
            for j in range(_LANES):
                if j + 2 < _LANES:
                    fire_vec(uvec, pvec, j + 2, (j + 2) % 4)
                else:
                    jn = j + 2 - _LANES

                    @pl.when(g + 1 < n_groups)
                    def _():
                        uvec_n = uidx_s[pl.ds((g + 1) * _LANES, _LANES)]
                        pvec_n = pidx_s[pl.ds((g + 1) * _LANES, _LANES)]
                        fire_vec(uvec_n, pvec_n, jn, jn)

                drain(j)
                res = compute(j, res)
            res = 1.0 / (1.0 + jnp.exp(-res))
            outv[pl.ds(g * _LANES, _LANES)] = res
            return 0

        lax.fori_loop(0, n_groups, group_body, 0)

        pltpu.sync_copy(outv, out_hbm.at[pl.ds(base, b_per_w)])

    return k


@jax.jit
def kernel(inputs, user_table, product_table):
    batch = inputs.shape[0]
    rows = user_table.shape[0]
    uidx = inputs[:, 0].astype(jnp.int32)
    pidx = inputs[:, 1].astype(jnp.int32)
    ut_t = user_table.T
    pt_t = product_table.T
    tail_start = (rows // _BLK) * _BLK
    pad = _BLK - (rows - tail_start)
    ut_tail = jnp.pad(ut_t[:, tail_start:], ((0, 0), (0, pad)))
    pt_tail = jnp.pad(pt_t[:, tail_start:], ((0, 0), (0, pad)))
    k = _make_kernel(batch, rows)
    return k(uidx, pidx, ut_t, pt_t, ut_tail, pt_tail)
